# Initial kernel scaffold; baseline (speedup 1.0000x reference)
import jax
import jax.numpy as jnp
from jax.experimental import pallas as pl


def kernel(inputs, pos_encoding, spatial_encoding):
    raise NotImplementedError("write your pallas kernel here")


# speedup vs baseline: 28.8291x; 28.8291x over previous
import jax
import jax.numpy as jnp
from jax.experimental import pallas as pl

_BLK = 512


def _body(x_ref, p_ref, s_ref, o_ref):
    spatial = jnp.sum(s_ref[...], axis=0, keepdims=True) * 0.2
    o_ref[...] = x_ref[...] + p_ref[...] + spatial[None]


def kernel(inputs, pos_encoding, spatial_encoding):
    b, s, d = inputs.shape
    pos = pos_encoding[0]
    grid = (b, s // _BLK)
    return pl.pallas_call(
        _body,
        grid=grid,
        in_specs=[
            pl.BlockSpec((1, _BLK, d), lambda i, j: (i, j, 0)),
            pl.BlockSpec((_BLK, d), lambda i, j: (j, 0)),
            pl.BlockSpec((5, d), lambda i, j: (0, 0)),
        ],
        out_specs=pl.BlockSpec((1, _BLK, d), lambda i, j: (i, j, 0)),
        out_shape=jax.ShapeDtypeStruct((b, s, d), inputs.dtype),
    )(inputs, pos, spatial_encoding)
